# Initial kernel scaffold; baseline (speedup 1.0000x reference)
import jax
import jax.numpy as jnp
from jax.experimental import pallas as pl


def kernel(x, edge_index, mask_x_position, emb, W1_l, W1_r, b1, W2_l, W2_r, b2):
    raise NotImplementedError("write your pallas kernel here")


# speedup vs baseline: 3.6468x; 3.6468x over previous
import functools
import jax
import jax.numpy as jnp
from jax import lax
from jax.experimental import pallas as pl
from jax.experimental.pallas import tpu as pltpu
from jax.experimental.pallas import tpu_sc as plsc

N = 10000
VOCAB = 10000
D = 128
PAD_N = 10240
D_PAD = 10240
E = 320000
CHUNK = 128
NCHUNK = 80
E_PAD = 32 * NCHUNK * CHUNK
TRASH = PAD_N - 1
NMASK = 1024

NC, NS = 2, 16
NW = NC * NS
ROWS_W = PAD_N // NW
ROWS_S = PAD_N // NS
MROWS_W = NMASK // NS

_mesh = plsc.VectorSubcoreMesh(core_axis_name="c", subcore_axis_name="s")


@functools.partial(
    pl.kernel,
    out_type=jax.ShapeDtypeStruct((PAD_N, D), jnp.float32),
    mesh=_mesh,
    scratch_types=[
        pltpu.VMEM((ROWS_W,), jnp.int32),
        pltpu.VMEM((ROWS_W, D), jnp.float32),
        pltpu.SemaphoreType.DMA,
    ],
)
def _emb_gather(emb_hbm, xid_hbm, h0_hbm, idx_v, rows_v, sem):
    wid = lax.axis_index("s") * NC + lax.axis_index("c")
    base = wid * ROWS_W
    pltpu.sync_copy(xid_hbm.at[pl.ds(base, ROWS_W)], idx_v)
    pltpu.async_copy(emb_hbm.at[idx_v], rows_v, sem).wait()
    pltpu.sync_copy(rows_v, h0_hbm.at[pl.ds(base, ROWS_W)])


@functools.partial(
    pl.kernel,
    out_type=(
        jax.ShapeDtypeStruct((NC, PAD_N, D), jnp.float32),
        jax.ShapeDtypeStruct((NC, NS, PAD_N), jnp.float32),
    ),
    mesh=_mesh,
    scratch_types=[
        pltpu.VMEM((NCHUNK, CHUNK), jnp.int32),
        pltpu.VMEM((NCHUNK, CHUNK), jnp.int32),
        pltpu.VMEM((CHUNK, D), jnp.float32),
        pltpu.VMEM((PAD_N,), jnp.float32),
        pltpu.VMEM_SHARED((PAD_N, D), jnp.float32),
        pltpu.SemaphoreType.DMA,
        pltpu.SemaphoreType.DMA,
    ],
    compiler_params=pltpu.CompilerParams(needs_layout_passes=False),
)
def _agg1(h_hbm, srcs_hbm, dsts_hbm, zrows_hbm, zn_hbm, p_out, cnt_out,
          src_v, dst_v, rows_v, cnt_v, agg_sh, gsem, ssem):
    c = lax.axis_index("c")
    s = lax.axis_index("s")
    wid = s * NC + c
    pltpu.sync_copy(zrows_hbm, agg_sh.at[pl.ds(s * ROWS_S, ROWS_S)])
    pltpu.sync_copy(zn_hbm, cnt_v)
    pltpu.sync_copy(srcs_hbm.at[wid], src_v)
    pltpu.sync_copy(dsts_hbm.at[wid], dst_v)
    plsc.subcore_barrier()

    ones16 = jnp.ones((16,), jnp.float32)

    def body(j, carry):
        pltpu.async_copy(h_hbm.at[src_v.at[j]], rows_v, gsem).wait()
        pltpu.async_copy(rows_v, agg_sh.at[dst_v.at[j]], ssem, add=True).wait()
        for i in range(CHUNK // 16):
            d16 = dst_v[j, pl.ds(i * 16, 16)]
            plsc.addupdate_scatter(cnt_v, (d16,), ones16)
        return carry

    lax.fori_loop(0, NCHUNK, body, 0)
    plsc.subcore_barrier()
    sl = pl.ds(s * ROWS_S, ROWS_S)
    pltpu.sync_copy(agg_sh.at[sl], p_out.at[c, sl])
    pltpu.sync_copy(cnt_v, cnt_out.at[c, s])


@functools.partial(
    pl.kernel,
    out_type=(
        jax.ShapeDtypeStruct((NC, NMASK, D), jnp.float32),
        jax.ShapeDtypeStruct((NMASK, D), jnp.float32),
        jax.ShapeDtypeStruct((NMASK, D), jnp.float32),
    ),
    mesh=_mesh,
    scratch_types=[
        pltpu.VMEM((NCHUNK, CHUNK), jnp.int32),
        pltpu.VMEM((NCHUNK, CHUNK), jnp.int32),
        pltpu.VMEM((CHUNK, D), jnp.float32),
        pltpu.VMEM((MROWS_W,), jnp.int32),
        pltpu.VMEM_SHARED((PAD_N, D), jnp.float32),
        pltpu.SemaphoreType.DMA,
        pltpu.SemaphoreType.DMA,
    ],
)
def _agg2(h_hbm, srcs_hbm, dsts_hbm, zrows_hbm, mask_hbm, icn_hbm,
          aggm_out, h1m_out, icm_out,
          src_v, dst_v, rows_v, mask_v, agg_sh, gsem, ssem):
    c = lax.axis_index("c")
    s = lax.axis_index("s")
    wid = s * NC + c
    pltpu.sync_copy(zrows_hbm, agg_sh.at[pl.ds(s * ROWS_S, ROWS_S)])
    pltpu.sync_copy(srcs_hbm.at[wid], src_v)
    pltpu.sync_copy(dsts_hbm.at[wid], dst_v)
    plsc.subcore_barrier()

    def body(j, carry):
        pltpu.async_copy(h_hbm.at[src_v.at[j]], rows_v, gsem).wait()
        pltpu.async_copy(rows_v, agg_sh.at[dst_v.at[j]], ssem, add=True).wait()
        return carry

    lax.fori_loop(0, NCHUNK, body, 0)
    plsc.subcore_barrier()

    mb = s * MROWS_W
    mrows_v = rows_v.at[pl.ds(0, MROWS_W)]
    pltpu.sync_copy(mask_hbm.at[pl.ds(mb, MROWS_W)], mask_v)
    pltpu.async_copy(agg_sh.at[mask_v], mrows_v, gsem).wait()
    pltpu.sync_copy(mrows_v, aggm_out.at[c, pl.ds(mb, MROWS_W)])

    @pl.when(c == 0)
    def _():
        pltpu.async_copy(h_hbm.at[mask_v], mrows_v, gsem).wait()
        pltpu.sync_copy(mrows_v, h1m_out.at[pl.ds(mb, MROWS_W)])

    @pl.when(c == 1)
    def _():
        pltpu.async_copy(icn_hbm.at[mask_v], mrows_v, gsem).wait()
        pltpu.sync_copy(mrows_v, icm_out.at[pl.ds(mb, MROWS_W)])


BLK1 = 1024

def _layer1_body(p0_ref, p1_ref, cnt_ref, h0_ref, wl_ref, wr_ref, b1_ref,
                 h1_ref, inv_ref):
    cnt = jnp.sum(cnt_ref[...], axis=1, keepdims=True)
    inv = 1.0 / jnp.maximum(cnt, 1.0)
    agg = (p0_ref[...] + p1_ref[...]) * inv
    h1 = (jnp.dot(agg, wl_ref[...], preferred_element_type=jnp.float32)
          + jnp.dot(h0_ref[...], wr_ref[...], preferred_element_type=jnp.float32)
          + b1_ref[...])
    h1_ref[...] = jnp.maximum(h1, 0.0)
    inv_ref[...] = jnp.broadcast_to(inv, (BLK1, D))


def _layer1_tc(p0, p1, cnt_t, h0, wl, wr, b1):
    grid = (PAD_N // BLK1,)
    return pl.pallas_call(
        _layer1_body,
        grid=grid,
        in_specs=[
            pl.BlockSpec((BLK1, D), lambda i: (i, 0)),
            pl.BlockSpec((BLK1, D), lambda i: (i, 0)),
            pl.BlockSpec((BLK1, NW), lambda i: (i, 0)),
            pl.BlockSpec((BLK1, D), lambda i: (i, 0)),
            pl.BlockSpec((D, D), lambda i: (0, 0)),
            pl.BlockSpec((D, D), lambda i: (0, 0)),
            pl.BlockSpec((1, D), lambda i: (0, 0)),
        ],
        out_specs=(
            pl.BlockSpec((BLK1, D), lambda i: (i, 0)),
            pl.BlockSpec((BLK1, D), lambda i: (i, 0)),
        ),
        out_shape=(
            jax.ShapeDtypeStruct((PAD_N, D), jnp.float32),
            jax.ShapeDtypeStruct((PAD_N, D), jnp.float32),
        ),
    )(p0, p1, cnt_t, h0, wl, wr, b1)


BLK2 = 128

def _out_body(a0_ref, a1_ref, h1m_ref, icm_ref, wl_ref, wr_ref, b2_ref, out_ref):
    agg = (a0_ref[...] + a1_ref[...]) * icm_ref[:, 0:1]
    logits = (jnp.dot(agg, wl_ref[...], preferred_element_type=jnp.float32)
              + jnp.dot(h1m_ref[...], wr_ref[...], preferred_element_type=jnp.float32)
              + b2_ref[...])
    m = jnp.max(logits, axis=1, keepdims=True)
    lse = jnp.log(jnp.sum(jnp.exp(logits - m), axis=1, keepdims=True))
    out_ref[...] = logits - m - lse


def _out_tc(a0, a1, h1m, icm, wl, wr, b2p):
    grid = (NMASK // BLK2,)
    return pl.pallas_call(
        _out_body,
        grid=grid,
        in_specs=[
            pl.BlockSpec((BLK2, D), lambda i: (i, 0)),
            pl.BlockSpec((BLK2, D), lambda i: (i, 0)),
            pl.BlockSpec((BLK2, D), lambda i: (i, 0)),
            pl.BlockSpec((BLK2, D), lambda i: (i, 0)),
            pl.BlockSpec((D, D_PAD), lambda i: (0, 0)),
            pl.BlockSpec((D, D_PAD), lambda i: (0, 0)),
            pl.BlockSpec((1, D_PAD), lambda i: (0, 0)),
        ],
        out_specs=pl.BlockSpec((BLK2, D_PAD), lambda i: (i, 0)),
        out_shape=jax.ShapeDtypeStruct((NMASK, D_PAD), jnp.float32),
    )(a0, a1, h1m, icm, wl, wr, b2p)


def kernel(x, edge_index, mask_x_position, emb, W1_l, W1_r, b1, W2_l, W2_r, b2):
    f32 = jnp.float32
    i32 = jnp.int32
    x_pad = jnp.concatenate(
        [x[:, 0].astype(i32), jnp.zeros((PAD_N - N,), i32)])
    src_p = jnp.concatenate(
        [edge_index[0].astype(i32), jnp.zeros((E_PAD - E,), i32)]
    ).reshape(NW, NCHUNK, CHUNK)
    dst_p = jnp.concatenate(
        [edge_index[1].astype(i32), jnp.full((E_PAD - E,), TRASH, i32)]
    ).reshape(NW, NCHUNK, CHUNK)
    zrows = jnp.zeros((ROWS_S, D), f32)
    zn = jnp.zeros((PAD_N,), f32)
    mask = mask_x_position.astype(i32)

    w1l = W1_l.T
    w1r = W1_r.T
    b1r = b1.reshape(1, D)
    w2l = jnp.pad(W2_l, ((0, D_PAD - VOCAB), (0, 0))).T
    w2r = jnp.pad(W2_r, ((0, D_PAD - VOCAB), (0, 0))).T
    b2p = jnp.pad(b2, (0, D_PAD - VOCAB), constant_values=-1e30).reshape(1, D_PAD)

    h0 = _emb_gather(emb, x_pad)
    p, cnt = _agg1(h0, src_p, dst_p, zrows, zn)
    cnt_t = cnt.reshape(NW, PAD_N).T
    h1, inv = _layer1_tc(p[0], p[1], cnt_t, h0, w1l, w1r, b1r)
    aggm, h1m, icm = _agg2(h1, src_p, dst_p, zrows, mask, inv)
    out = _out_tc(aggm[0], aggm[1], h1m, icm, w2l, w2r, b2p)
    return out[:, :VOCAB]


# speedup vs baseline: 3.9802x; 1.0914x over previous
import functools
import jax
import jax.numpy as jnp
from jax import lax
from jax.experimental import pallas as pl
from jax.experimental.pallas import tpu as pltpu
from jax.experimental.pallas import tpu_sc as plsc

N = 10000
VOCAB = 10000
D = 128
PAD_N = 10240
D_PAD = 10240
E = 320000
CHUNK = 128
NCHUNK = 80
E_PAD = 32 * NCHUNK * CHUNK
TRASH = PAD_N - 1
NMASK = 1024

NC, NS = 2, 16
NW = NC * NS
ROWS_W = PAD_N // NW
ROWS_S = PAD_N // NS
MROWS_W = NMASK // NS

_mesh = plsc.VectorSubcoreMesh(core_axis_name="c", subcore_axis_name="s")


@functools.partial(
    pl.kernel,
    out_type=(
        jax.ShapeDtypeStruct((PAD_N, D), jnp.float32),
        jax.ShapeDtypeStruct((NC, NS, PAD_N), jnp.float32),
    ),
    mesh=_mesh,
    scratch_types=[
        pltpu.VMEM((ROWS_W,), jnp.int32),
        pltpu.VMEM((ROWS_W, D), jnp.float32),
        pltpu.VMEM((NCHUNK, CHUNK), jnp.int32),
        pltpu.VMEM((PAD_N,), jnp.float32),
        pltpu.SemaphoreType.DMA,
    ],
    compiler_params=pltpu.CompilerParams(needs_layout_passes=False),
)
def _emb_gather(emb_hbm, xid_hbm, dsts_hbm, zn_hbm, h0_hbm, cnt_out,
                idx_v, rows_v, dst_v, cnt_v, sem):
    c = lax.axis_index("c")
    s = lax.axis_index("s")
    wid = s * NC + c
    base = wid * ROWS_W
    pltpu.sync_copy(xid_hbm.at[pl.ds(base, ROWS_W)], idx_v)
    pltpu.async_copy(emb_hbm.at[idx_v], rows_v, sem)
    pltpu.sync_copy(dsts_hbm.at[wid], dst_v)
    pltpu.sync_copy(zn_hbm, cnt_v)
    ones16 = jnp.ones((16,), jnp.float32)

    def body(j, carry):
        for i in range(CHUNK // 16):
            d16 = dst_v[j, pl.ds(i * 16, 16)]
            plsc.addupdate_scatter(cnt_v, (d16,), ones16)
        return carry

    lax.fori_loop(0, NCHUNK, body, 0)
    pltpu.make_async_copy(emb_hbm.at[pl.ds(0, ROWS_W)], rows_v, sem).wait()
    pltpu.sync_copy(rows_v, h0_hbm.at[pl.ds(base, ROWS_W)])
    pltpu.sync_copy(cnt_v, cnt_out.at[c, s])


NPHASE = 2
PCHUNK = NCHUNK // NPHASE


def _wait(hbm, sem, buf):
    pltpu.make_async_copy(hbm.at[pl.ds(0, CHUNK)], buf, sem).wait()


def _edge_pipeline(h_hbm, srcs_hbm, dsts_hbm, wid, src_v, dst_v,
                   rows_a, rows_b, agg_sh, gsem, ssem):
    for p in range(NPHASE):
        pltpu.sync_copy(srcs_hbm.at[wid, pl.ds(p * PCHUNK, PCHUNK)], src_v)
        pltpu.sync_copy(dsts_hbm.at[wid, pl.ds(p * PCHUNK, PCHUNK)], dst_v)
        pltpu.async_copy(h_hbm.at[src_v.at[0]], rows_a, gsem)

        def body(j, carry):
            e = 2 * j
            _wait(h_hbm, gsem, rows_a)

            @pl.when(j > 0)
            def _():
                _wait(h_hbm, ssem, rows_b)

            pltpu.async_copy(h_hbm.at[src_v.at[e + 1]], rows_b, gsem)
            pltpu.async_copy(rows_a, agg_sh.at[dst_v.at[e]], ssem, add=True)
            _wait(h_hbm, gsem, rows_b)
            _wait(h_hbm, ssem, rows_a)

            @pl.when(j < PCHUNK // 2 - 1)
            def _():
                pltpu.async_copy(h_hbm.at[src_v.at[e + 2]], rows_a, gsem)

            pltpu.async_copy(rows_b, agg_sh.at[dst_v.at[e + 1]], ssem, add=True)
            return carry

        lax.fori_loop(0, PCHUNK // 2, body, 0)
        _wait(h_hbm, ssem, rows_b)


@functools.partial(
    pl.kernel,
    out_type=jax.ShapeDtypeStruct((NC, PAD_N, D), jnp.float32),
    mesh=_mesh,
    scratch_types=[
        pltpu.VMEM((PCHUNK, CHUNK), jnp.int32),
        pltpu.VMEM((PCHUNK, CHUNK), jnp.int32),
        pltpu.VMEM((CHUNK, D), jnp.float32),
        pltpu.VMEM((CHUNK, D), jnp.float32),
        pltpu.VMEM_SHARED((PAD_N, D), jnp.float32),
        pltpu.SemaphoreType.DMA,
        pltpu.SemaphoreType.DMA,
    ],
)
def _agg1(h_hbm, srcs_hbm, dsts_hbm, zrows_hbm, p_out,
          src_v, dst_v, rows_a, rows_b, agg_sh, gsem, ssem):
    c = lax.axis_index("c")
    s = lax.axis_index("s")
    wid = s * NC + c
    pltpu.sync_copy(zrows_hbm, agg_sh.at[pl.ds(s * ROWS_S, ROWS_S)])
    plsc.subcore_barrier()
    _edge_pipeline(h_hbm, srcs_hbm, dsts_hbm, wid, src_v, dst_v,
                   rows_a, rows_b, agg_sh, gsem, ssem)
    plsc.subcore_barrier()
    sl = pl.ds(s * ROWS_S, ROWS_S)
    pltpu.sync_copy(agg_sh.at[sl], p_out.at[c, sl])


@functools.partial(
    pl.kernel,
    out_type=(
        jax.ShapeDtypeStruct((NC, NMASK, D), jnp.float32),
        jax.ShapeDtypeStruct((NMASK, D), jnp.float32),
        jax.ShapeDtypeStruct((NMASK, D), jnp.float32),
    ),
    mesh=_mesh,
    scratch_types=[
        pltpu.VMEM((PCHUNK, CHUNK), jnp.int32),
        pltpu.VMEM((PCHUNK, CHUNK), jnp.int32),
        pltpu.VMEM((CHUNK, D), jnp.float32),
        pltpu.VMEM((CHUNK, D), jnp.float32),
        pltpu.VMEM((MROWS_W,), jnp.int32),
        pltpu.VMEM_SHARED((PAD_N, D), jnp.float32),
        pltpu.SemaphoreType.DMA,
        pltpu.SemaphoreType.DMA,
    ],
)
def _agg2(h_hbm, srcs_hbm, dsts_hbm, zrows_hbm, mask_hbm, icn_hbm,
          aggm_out, h1m_out, icm_out,
          src_v, dst_v, rows_a, rows_b, mask_v, agg_sh, gsem, ssem):
    c = lax.axis_index("c")
    s = lax.axis_index("s")
    wid = s * NC + c
    pltpu.sync_copy(zrows_hbm, agg_sh.at[pl.ds(s * ROWS_S, ROWS_S)])
    plsc.subcore_barrier()
    _edge_pipeline(h_hbm, srcs_hbm, dsts_hbm, wid, src_v, dst_v,
                   rows_a, rows_b, agg_sh, gsem, ssem)
    plsc.subcore_barrier()

    mb = s * MROWS_W
    mrows_v = rows_a.at[pl.ds(0, MROWS_W)]
    pltpu.sync_copy(mask_hbm.at[pl.ds(mb, MROWS_W)], mask_v)
    pltpu.async_copy(agg_sh.at[mask_v], mrows_v, gsem).wait()
    pltpu.sync_copy(mrows_v, aggm_out.at[c, pl.ds(mb, MROWS_W)])

    @pl.when(c == 0)
    def _():
        pltpu.async_copy(h_hbm.at[mask_v], mrows_v, gsem).wait()
        pltpu.sync_copy(mrows_v, h1m_out.at[pl.ds(mb, MROWS_W)])

    @pl.when(c == 1)
    def _():
        pltpu.async_copy(icn_hbm.at[mask_v], mrows_v, gsem).wait()
        pltpu.sync_copy(mrows_v, icm_out.at[pl.ds(mb, MROWS_W)])


BLK1 = 1024

def _layer1_body(p0_ref, p1_ref, cnt_ref, h0_ref, wl_ref, wr_ref, b1_ref,
                 h1_ref, inv_ref):
    cnt = jnp.sum(cnt_ref[...], axis=1, keepdims=True)
    inv = 1.0 / jnp.maximum(cnt, 1.0)
    agg = (p0_ref[...] + p1_ref[...]) * inv
    h1 = (jnp.dot(agg, wl_ref[...], preferred_element_type=jnp.float32)
          + jnp.dot(h0_ref[...], wr_ref[...], preferred_element_type=jnp.float32)
          + b1_ref[...])
    h1_ref[...] = jnp.maximum(h1, 0.0)
    inv_ref[...] = jnp.broadcast_to(inv, (BLK1, D))


def _layer1_tc(p0, p1, cnt_t, h0, wl, wr, b1):
    grid = (PAD_N // BLK1,)
    return pl.pallas_call(
        _layer1_body,
        grid=grid,
        in_specs=[
            pl.BlockSpec((BLK1, D), lambda i: (i, 0)),
            pl.BlockSpec((BLK1, D), lambda i: (i, 0)),
            pl.BlockSpec((BLK1, NW), lambda i: (i, 0)),
            pl.BlockSpec((BLK1, D), lambda i: (i, 0)),
            pl.BlockSpec((D, D), lambda i: (0, 0)),
            pl.BlockSpec((D, D), lambda i: (0, 0)),
            pl.BlockSpec((1, D), lambda i: (0, 0)),
        ],
        out_specs=(
            pl.BlockSpec((BLK1, D), lambda i: (i, 0)),
            pl.BlockSpec((BLK1, D), lambda i: (i, 0)),
        ),
        out_shape=(
            jax.ShapeDtypeStruct((PAD_N, D), jnp.float32),
            jax.ShapeDtypeStruct((PAD_N, D), jnp.float32),
        ),
    )(p0, p1, cnt_t, h0, wl, wr, b1)


BLK2 = 128

def _out_body(a0_ref, a1_ref, h1m_ref, icm_ref, wl_ref, wr_ref, b2_ref, out_ref):
    agg = (a0_ref[...] + a1_ref[...]) * icm_ref[:, 0:1]
    logits = (jnp.dot(agg, wl_ref[...], preferred_element_type=jnp.float32)
              + jnp.dot(h1m_ref[...], wr_ref[...], preferred_element_type=jnp.float32)
              + b2_ref[...])
    m = jnp.max(logits, axis=1, keepdims=True)
    lse = jnp.log(jnp.sum(jnp.exp(logits - m), axis=1, keepdims=True))
    out_ref[...] = logits - m - lse


def _out_tc(a0, a1, h1m, icm, wl, wr, b2p):
    grid = (NMASK // BLK2,)
    return pl.pallas_call(
        _out_body,
        grid=grid,
        in_specs=[
            pl.BlockSpec((BLK2, D), lambda i: (i, 0)),
            pl.BlockSpec((BLK2, D), lambda i: (i, 0)),
            pl.BlockSpec((BLK2, D), lambda i: (i, 0)),
            pl.BlockSpec((BLK2, D), lambda i: (i, 0)),
            pl.BlockSpec((D, D_PAD), lambda i: (0, 0)),
            pl.BlockSpec((D, D_PAD), lambda i: (0, 0)),
            pl.BlockSpec((1, D_PAD), lambda i: (0, 0)),
        ],
        out_specs=pl.BlockSpec((BLK2, D_PAD), lambda i: (i, 0)),
        out_shape=jax.ShapeDtypeStruct((NMASK, D_PAD), jnp.float32),
    )(a0, a1, h1m, icm, wl, wr, b2p)


def kernel(x, edge_index, mask_x_position, emb, W1_l, W1_r, b1, W2_l, W2_r, b2):
    f32 = jnp.float32
    i32 = jnp.int32
    x_pad = jnp.concatenate(
        [x[:, 0].astype(i32), jnp.zeros((PAD_N - N,), i32)])
    src_p = jnp.concatenate(
        [edge_index[0].astype(i32), jnp.zeros((E_PAD - E,), i32)]
    ).reshape(NW, NCHUNK, CHUNK)
    dst_p = jnp.concatenate(
        [edge_index[1].astype(i32), jnp.full((E_PAD - E,), TRASH, i32)]
    ).reshape(NW, NCHUNK, CHUNK)
    zrows = jnp.zeros((ROWS_S, D), f32)
    zn = jnp.zeros((PAD_N,), f32)
    mask = mask_x_position.astype(i32)

    w1l = W1_l.T
    w1r = W1_r.T
    b1r = b1.reshape(1, D)
    w2l = jnp.pad(W2_l, ((0, D_PAD - VOCAB), (0, 0))).T
    w2r = jnp.pad(W2_r, ((0, D_PAD - VOCAB), (0, 0))).T
    b2p = jnp.pad(b2, (0, D_PAD - VOCAB), constant_values=-1e30).reshape(1, D_PAD)

    h0, cnt = _emb_gather(emb, x_pad, dst_p, zn)
    p = _agg1(h0, src_p, dst_p, zrows)
    cnt_t = cnt.reshape(NW, PAD_N).T
    h1, inv = _layer1_tc(p[0], p[1], cnt_t, h0, w1l, w1r, b1r)
    aggm, h1m, icm = _agg2(h1, src_p, dst_p, zrows, mask, inv)
    out = _out_tc(aggm[0], aggm[1], h1m, icm, w2l, w2r, b2p)
    return out[:, :VOCAB]
